# baseline (speedup 1.0000x reference)
import functools

import jax
import jax.numpy as jnp
from jax import lax
from jax.experimental import pallas as pl
from jax.experimental.pallas import tpu as pltpu
from jax.experimental.pallas import tpu_sc as plsc

_NC = 2
_NS = 16
_L = 16
_NW = _NC * _NS
_NBINS = 4096
_H = 2048
_WIDTH = 2048
_ROWS_PER_W = _H // _NW
_CH_ROWS = 2
_NCHUNK = _ROWS_PER_W // _CH_ROWS

_mesh = plsc.VectorSubcoreMesh(
    core_axis_name="c", subcore_axis_name="s", num_cores=_NC, num_subcores=_NS
)


@functools.partial(
    pl.kernel,
    out_type=jax.ShapeDtypeStruct((_NW, _NBINS), jnp.int32),
    mesh=_mesh,
    compiler_params=pltpu.CompilerParams(needs_layout_passes=False),
    scratch_types=[
        pltpu.VMEM((2, 3, _CH_ROWS, _WIDTH), jnp.int32),
        pltpu.VMEM((_L * _NBINS,), jnp.int32),
        pltpu.SemaphoreType.DMA,
        pltpu.SemaphoreType.DMA,
    ],
)
def _hist(img_hbm, out_hbm, buf, hist, sem0, sem1):
    wid = lax.axis_index("s") * _NC + lax.axis_index("c")
    r0 = wid * _ROWS_PER_W
    iota = lax.iota(jnp.int32, _L)
    lane_base = iota * _NBINS
    zeros = iota * 0
    ones = zeros + 1

    def zbody(i, carry):
        for u in range(8):
            hist[pl.ds(i * (8 * _L) + u * _L, _L)] = zeros
        return carry

    lax.fori_loop(0, (_L * _NBINS) // (8 * _L), zbody, 0)

    sems = (sem0, sem1)

    def start(j):
        rj = r0 + j * _CH_ROWS
        return [
            pltpu.async_copy(
                img_hbm.at[c, pl.ds(rj, _CH_ROWS), :], buf.at[j % 2, c], sems[j % 2]
            )
            for c in range(3)
        ]

    descs = [start(0), None]
    for j in range(_NCHUNK):
        if j + 1 < _NCHUNK:
            descs[(j + 1) % 2] = start(j + 1)
        for d in descs[j % 2]:
            d.wait()
        _U = 8
        for rr in range(_CH_ROWS):
            def it(i, carry):
                xs = []
                for u in range(_U):
                    o = i * (_U * _L) + u * _L
                    xs.append(
                        (
                            buf[j % 2, 0, rr, pl.ds(o, _L)],
                            buf[j % 2, 1, rr, pl.ds(o, _L)],
                            buf[j % 2, 2, rr, pl.ds(o, _L)],
                        )
                    )
                addrs = [
                    lane_base + (((xr & 0xF0) << 4) | (xg & 0xF0) | (xb >> 4))
                    for (xb, xg, xr) in xs
                ]
                for a in addrs:
                    plsc.addupdate_scatter(hist, [a], ones)
                return carry

            lax.fori_loop(0, _WIDTH // (_U * _L), it, 0)

    def rbody(i, carry):
        o = i * _L
        acc = hist[pl.ds(o, _L)]
        for l in range(1, _L):
            acc = acc + hist[pl.ds(l * _NBINS + o, _L)]
        hist[pl.ds(o, _L)] = acc
        return carry

    lax.fori_loop(0, _NBINS // _L, rbody, 0)
    pltpu.sync_copy(hist.at[pl.ds(0, _NBINS)], out_hbm.at[wid])


_BLK = _NBINS // _NW


@functools.partial(
    pl.kernel,
    out_type=jax.ShapeDtypeStruct((_NBINS + _L,), jnp.int32),
    mesh=_mesh,
    compiler_params=pltpu.CompilerParams(needs_layout_passes=False),
    scratch_types=[
        pltpu.VMEM((_NW, _BLK), jnp.int32),
        pltpu.VMEM((_BLK,), jnp.int32),
        pltpu.VMEM((_L,), jnp.int32),
        pltpu.SemaphoreType.DMA,
    ],
)
def _reduce(parts_hbm, out_hbm, buf, acc, sent, sem):
    wid = lax.axis_index("s") * _NC + lax.axis_index("c")
    o = wid * _BLK
    descs = [
        pltpu.async_copy(parts_hbm.at[t, pl.ds(o, _BLK)], buf.at[t], sem)
        for t in range(_NW)
    ]
    for d in descs:
        d.wait()
    for i in range(_BLK // _L):
        acc16 = buf[0, pl.ds(i * _L, _L)]
        for t in range(1, _NW):
            acc16 = acc16 + buf[t, pl.ds(i * _L, _L)]
        acc[pl.ds(i * _L, _L)] = acc16
    pltpu.sync_copy(acc, out_hbm.at[pl.ds(o, _BLK)])

    @pl.when(wid == 0)
    def _():
        sent[...] = (lax.iota(jnp.int32, _L) == 0).astype(jnp.int32)
        pltpu.sync_copy(sent, out_hbm.at[pl.ds(_NBINS, _L)])


@jax.jit
def kernel(img):
    planar = jnp.transpose(img.astype(jnp.int32), (2, 0, 1))
    parts = _hist(planar)
    full = _reduce(parts)
    return full[: _NBINS + 1]


# speedup vs baseline: 1.0560x; 1.0560x over previous
import functools

import jax
import jax.numpy as jnp
from jax import lax
from jax.experimental import pallas as pl
from jax.experimental.pallas import tpu as pltpu
from jax.experimental.pallas import tpu_sc as plsc

_NC = 2
_NS = 16
_L = 16
_NW = _NC * _NS
_NBINS = 4096
_H = 2048
_WIDTH = 2048
_ROWS_PER_W = _H // _NW
_CH_ROWS = 4
_NCHUNK = _ROWS_PER_W // _CH_ROWS

_mesh = plsc.VectorSubcoreMesh(
    core_axis_name="c", subcore_axis_name="s", num_cores=_NC, num_subcores=_NS
)


@functools.partial(
    pl.kernel,
    out_type=jax.ShapeDtypeStruct((_NW, _NBINS), jnp.int32),
    mesh=_mesh,
    compiler_params=pltpu.CompilerParams(needs_layout_passes=False),
    scratch_types=[
        pltpu.VMEM((2, 3, _CH_ROWS, _WIDTH), jnp.int32),
        pltpu.VMEM((_L * _NBINS,), jnp.int32),
        pltpu.SemaphoreType.DMA,
        pltpu.SemaphoreType.DMA,
    ],
)
def _hist(img_hbm, out_hbm, buf, hist, sem0, sem1):
    wid = lax.axis_index("s") * _NC + lax.axis_index("c")
    r0 = wid * _ROWS_PER_W
    iota = lax.iota(jnp.int32, _L)
    lane_base = iota * _NBINS
    zeros = iota * 0
    ones = zeros + 1

    def zbody(i, carry):
        for u in range(8):
            hist[pl.ds(i * (8 * _L) + u * _L, _L)] = zeros
        return carry

    lax.fori_loop(0, (_L * _NBINS) // (8 * _L), zbody, 0)

    sems = (sem0, sem1)

    def start(j):
        rj = r0 + j * _CH_ROWS
        return [
            pltpu.async_copy(
                img_hbm.at[c, pl.ds(rj, _CH_ROWS), :], buf.at[j % 2, c], sems[j % 2]
            )
            for c in range(3)
        ]

    descs = [start(0), None]
    for j in range(_NCHUNK):
        if j + 1 < _NCHUNK:
            descs[(j + 1) % 2] = start(j + 1)
        for d in descs[j % 2]:
            d.wait()
        _U = 8
        for rr in range(_CH_ROWS):
            def it(i, carry):
                xs = []
                for u in range(_U):
                    o = i * (_U * _L) + u * _L
                    xs.append(
                        (
                            buf[j % 2, 0, rr, pl.ds(o, _L)],
                            buf[j % 2, 1, rr, pl.ds(o, _L)],
                            buf[j % 2, 2, rr, pl.ds(o, _L)],
                        )
                    )
                addrs = [
                    lane_base + (((xr & 0xF0) << 4) | (xg & 0xF0) | (xb >> 4))
                    for (xb, xg, xr) in xs
                ]
                for a in addrs:
                    plsc.addupdate_scatter(hist, [a], ones)
                return carry

            lax.fori_loop(0, _WIDTH // (_U * _L), it, 0)

    def rbody(i, carry):
        for u in range(2):
            o = i * (2 * _L) + u * _L
            vs = [hist[pl.ds(l * _NBINS + o, _L)] for l in range(_L)]
            while len(vs) > 1:
                vs = [a + b for a, b in zip(vs[0::2], vs[1::2])]
            hist[pl.ds(o, _L)] = vs[0]
        return carry

    lax.fori_loop(0, _NBINS // (2 * _L), rbody, 0)
    pltpu.sync_copy(hist.at[pl.ds(0, _NBINS)], out_hbm.at[wid])


_BLK = _NBINS // _NW


@functools.partial(
    pl.kernel,
    out_type=jax.ShapeDtypeStruct((_NBINS + _L,), jnp.int32),
    mesh=_mesh,
    compiler_params=pltpu.CompilerParams(needs_layout_passes=False),
    scratch_types=[
        pltpu.VMEM((_NW, _BLK), jnp.int32),
        pltpu.VMEM((_BLK,), jnp.int32),
        pltpu.VMEM((_L,), jnp.int32),
        pltpu.SemaphoreType.DMA,
    ],
)
def _reduce(parts_hbm, out_hbm, buf, acc, sent, sem):
    wid = lax.axis_index("s") * _NC + lax.axis_index("c")
    o = wid * _BLK
    descs = [
        pltpu.async_copy(parts_hbm.at[t, pl.ds(o, _BLK)], buf.at[t], sem)
        for t in range(_NW)
    ]
    for d in descs:
        d.wait()
    for i in range(_BLK // _L):
        acc16 = buf[0, pl.ds(i * _L, _L)]
        for t in range(1, _NW):
            acc16 = acc16 + buf[t, pl.ds(i * _L, _L)]
        acc[pl.ds(i * _L, _L)] = acc16
    pltpu.sync_copy(acc, out_hbm.at[pl.ds(o, _BLK)])

    @pl.when(wid == 0)
    def _():
        sent[...] = (lax.iota(jnp.int32, _L) == 0).astype(jnp.int32)
        pltpu.sync_copy(sent, out_hbm.at[pl.ds(_NBINS, _L)])


@jax.jit
def kernel(img):
    planar = jnp.transpose(img.astype(jnp.int32), (2, 0, 1))
    parts = _hist(planar)
    full = _reduce(parts)
    return full[: _NBINS + 1]


# speedup vs baseline: 1.0973x; 1.0390x over previous
import functools

import jax
import jax.numpy as jnp
from jax import lax
from jax.experimental import pallas as pl
from jax.experimental.pallas import tpu as pltpu
from jax.experimental.pallas import tpu_sc as plsc

_NC = 2
_NS = 16
_L = 16
_NW = _NC * _NS
_NBINS = 4096
_H = 2048
_WIDTH = 2048
_ROWS_PER_W = _H // _NW
_CH_ROWS = 4
_NCHUNK = _ROWS_PER_W // _CH_ROWS

_mesh = plsc.VectorSubcoreMesh(
    core_axis_name="c", subcore_axis_name="s", num_cores=_NC, num_subcores=_NS
)


@functools.partial(
    pl.kernel,
    out_type=jax.ShapeDtypeStruct((_NW, _NBINS), jnp.int32),
    mesh=_mesh,
    compiler_params=pltpu.CompilerParams(needs_layout_passes=False),
    scratch_types=[
        pltpu.VMEM((2, 3, _CH_ROWS, _WIDTH), jnp.int32),
        pltpu.VMEM((_L * _NBINS,), jnp.int32),
        pltpu.SemaphoreType.DMA,
        pltpu.SemaphoreType.DMA,
    ],
)
def _hist(img_hbm, out_hbm, buf, hist, sem0, sem1):
    wid = lax.axis_index("s") * _NC + lax.axis_index("c")
    r0 = wid * _ROWS_PER_W
    iota = lax.iota(jnp.int32, _L)
    lane_base = iota * _NBINS
    zeros = iota * 0
    ones = zeros + 1

    def zbody(i, carry):
        for u in range(8):
            hist[pl.ds(i * (8 * _L) + u * _L, _L)] = zeros
        return carry

    lax.fori_loop(0, (_L * _NBINS) // (8 * _L), zbody, 0)

    sems = (sem0, sem1)

    def start(j):
        rj = r0 + j * _CH_ROWS
        return [
            pltpu.async_copy(
                img_hbm.at[c, pl.ds(rj, _CH_ROWS), :], buf.at[j % 2, c], sems[j % 2]
            )
            for c in range(3)
        ]

    descs = [start(0), None]
    for j in range(_NCHUNK):
        if j + 1 < _NCHUNK:
            descs[(j + 1) % 2] = start(j + 1)
        for d in descs[j % 2]:
            d.wait()
        _U = 8
        for rr in range(_CH_ROWS):
            def it(i, carry):
                xs = []
                for u in range(_U):
                    o = i * (_U * _L) + u * _L
                    xs.append(
                        (
                            buf[j % 2, 0, rr, pl.ds(o, _L)],
                            buf[j % 2, 1, rr, pl.ds(o, _L)],
                            buf[j % 2, 2, rr, pl.ds(o, _L)],
                        )
                    )
                addrs = [
                    lane_base + (((xr & 0xF0) << 4) | (xg & 0xF0) | (xb >> 4))
                    for (xb, xg, xr) in xs
                ]
                for a in addrs:
                    plsc.addupdate_scatter(hist, [a], ones)
                return carry

            lax.fori_loop(0, _WIDTH // (_U * _L), it, 0)

    def rbody(i, carry):
        for u in range(2):
            o = i * (2 * _L) + u * _L
            vs = [hist[pl.ds(l * _NBINS + o, _L)] for l in range(_L)]
            while len(vs) > 1:
                vs = [a + b for a, b in zip(vs[0::2], vs[1::2])]
            hist[pl.ds(o, _L)] = vs[0]
        return carry

    lax.fori_loop(0, _NBINS // (2 * _L), rbody, 0)
    pltpu.sync_copy(hist.at[pl.ds(0, _NBINS)], out_hbm.at[wid])


def _reduce_tc_body(parts_ref, out_ref):
    s = jnp.sum(parts_ref[...], axis=0, keepdims=True)
    body = jnp.broadcast_to(s, (8, _NBINS))
    sent = (lax.broadcasted_iota(jnp.int32, (8, 8), 1) == 0).astype(jnp.int32)
    out_ref[...] = jnp.concatenate([body, sent], axis=1)


_reduce = pl.pallas_call(
    _reduce_tc_body,
    out_shape=jax.ShapeDtypeStruct((8, _NBINS + 8), jnp.int32),
)


@jax.jit
def kernel(img):
    planar = jnp.transpose(img.astype(jnp.int32), (2, 0, 1))
    parts = _hist(planar)
    full = _reduce(parts)
    return full[0, : _NBINS + 1]


# speedup vs baseline: 1.1305x; 1.0303x over previous
import functools

import jax
import jax.numpy as jnp
from jax import lax
from jax.experimental import pallas as pl
from jax.experimental.pallas import tpu as pltpu
from jax.experimental.pallas import tpu_sc as plsc

_NC = 2
_NS = 16
_L = 16
_NW = _NC * _NS
_NBINS = 4096
_H = 2048
_WIDTH = 2048
_ROWS_PER_W = _H // _NW
_CH_ROWS = 4
_NCHUNK = _ROWS_PER_W // _CH_ROWS

_mesh = plsc.VectorSubcoreMesh(
    core_axis_name="c", subcore_axis_name="s", num_cores=_NC, num_subcores=_NS
)


@functools.partial(
    pl.kernel,
    out_type=jax.ShapeDtypeStruct((_NW, _NBINS), jnp.int32),
    mesh=_mesh,
    compiler_params=pltpu.CompilerParams(needs_layout_passes=False),
    scratch_types=[
        pltpu.VMEM((2, 3, _CH_ROWS, _WIDTH), jnp.int32),
        pltpu.VMEM((_L * _NBINS,), jnp.int32),
        pltpu.SemaphoreType.DMA,
        pltpu.SemaphoreType.DMA,
    ],
)
def _hist(img_hbm, out_hbm, buf, hist, sem0, sem1):
    wid = lax.axis_index("s") * _NC + lax.axis_index("c")
    r0 = wid * _ROWS_PER_W
    iota = lax.iota(jnp.int32, _L)
    lane_base = iota * _NBINS
    zeros = iota * 0
    ones = zeros + 1

    sems = (sem0, sem1)

    def start(j):
        rj = r0 + j * _CH_ROWS
        return [
            pltpu.async_copy(
                img_hbm.at[c, pl.ds(rj, _CH_ROWS), :], buf.at[j % 2, c], sems[j % 2]
            )
            for c in range(3)
        ]

    descs = [start(0), None]
    descs[1] = start(1)

    def zbody(i, carry):
        for u in range(8):
            hist[pl.ds(i * (8 * _L) + u * _L, _L)] = zeros
        return carry

    lax.fori_loop(0, (_L * _NBINS) // (8 * _L), zbody, 0)
    for j in range(_NCHUNK):
        for d in descs[j % 2]:
            d.wait()
        _U = 8
        for rr in range(_CH_ROWS):
            def it(i, carry):
                xs = []
                for u in range(_U):
                    o = i * (_U * _L) + u * _L
                    xs.append(
                        (
                            buf[j % 2, 0, rr, pl.ds(o, _L)],
                            buf[j % 2, 1, rr, pl.ds(o, _L)],
                            buf[j % 2, 2, rr, pl.ds(o, _L)],
                        )
                    )
                addrs = [
                    lane_base + (((xr & 0xF0) << 4) | (xg & 0xF0) | (xb >> 4))
                    for (xb, xg, xr) in xs
                ]
                for a in addrs:
                    plsc.addupdate_scatter(hist, [a], ones)
                return carry

            lax.fori_loop(0, _WIDTH // (_U * _L), it, 0)

        if j + 2 < _NCHUNK:
            descs[j % 2] = start(j + 2)

    def rbody(i, carry):
        for u in range(2):
            o = i * (2 * _L) + u * _L
            vs = [hist[pl.ds(l * _NBINS + o, _L)] for l in range(_L)]
            while len(vs) > 1:
                vs = [a + b for a, b in zip(vs[0::2], vs[1::2])]
            hist[pl.ds(o, _L)] = vs[0]
        return carry

    lax.fori_loop(0, _NBINS // (2 * _L), rbody, 0)
    pltpu.sync_copy(hist.at[pl.ds(0, _NBINS)], out_hbm.at[wid])


def _reduce_tc_body(parts_ref, out_ref):
    s = jnp.sum(parts_ref[...], axis=0, keepdims=True)
    body = jnp.broadcast_to(s, (8, _NBINS))
    sent = (lax.broadcasted_iota(jnp.int32, (8, 8), 1) == 0).astype(jnp.int32)
    out_ref[...] = jnp.concatenate([body, sent], axis=1)


_reduce = pl.pallas_call(
    _reduce_tc_body,
    out_shape=jax.ShapeDtypeStruct((8, _NBINS + 8), jnp.int32),
)


@jax.jit
def kernel(img):
    planar = jnp.transpose(img.astype(jnp.int32), (2, 0, 1))
    parts = _hist(planar)
    full = _reduce(parts)
    return full[0, : _NBINS + 1]


# speedup vs baseline: 1.1661x; 1.0315x over previous
import functools

import jax
import jax.numpy as jnp
from jax import lax
from jax.experimental import pallas as pl
from jax.experimental.pallas import tpu as pltpu
from jax.experimental.pallas import tpu_sc as plsc

_NC = 2
_NS = 16
_L = 16
_NW = _NC * _NS
_NBINS = 4096
_H = 2048
_WIDTH = 2048
_ROWS_PER_W = _H // _NW
_CH_ROWS = 4
_NCHUNK = _ROWS_PER_W // _CH_ROWS

_mesh = plsc.VectorSubcoreMesh(
    core_axis_name="c", subcore_axis_name="s", num_cores=_NC, num_subcores=_NS
)


@functools.partial(
    pl.kernel,
    out_type=jax.ShapeDtypeStruct((_NW, _NBINS), jnp.int32),
    mesh=_mesh,
    compiler_params=pltpu.CompilerParams(needs_layout_passes=False),
    scratch_types=[
        pltpu.VMEM((2, 3, _CH_ROWS, _WIDTH), jnp.int32),
        pltpu.VMEM((_L * _NBINS,), jnp.int32),
        pltpu.SemaphoreType.DMA,
        pltpu.SemaphoreType.DMA,
    ],
)
def _hist(img_hbm, out_hbm, buf, hist, sem0, sem1):
    wid = lax.axis_index("s") * _NC + lax.axis_index("c")
    r0 = wid * _ROWS_PER_W
    iota = lax.iota(jnp.int32, _L)
    lane_base = iota * _NBINS
    zeros = iota * 0
    ones = zeros + 1

    sems = (sem0, sem1)

    def start(j):
        rj = r0 + j * _CH_ROWS
        return [
            pltpu.async_copy(
                img_hbm.at[c, pl.ds(rj, _CH_ROWS), :], buf.at[j % 2, c], sems[j % 2]
            )
            for c in range(3)
        ]

    descs = [start(0), None]
    descs[1] = start(1)

    def zbody(i, carry):
        for u in range(8):
            hist[pl.ds(i * (8 * _L) + u * _L, _L)] = zeros
        return carry

    lax.fori_loop(0, (_L * _NBINS) // (8 * _L), zbody, 0)
    for j in range(_NCHUNK):
        for d in descs[j % 2]:
            d.wait()
        _U = 16
        for rr in range(_CH_ROWS):
            def it(i, carry):
                xs = []
                for u in range(_U):
                    o = i * (_U * _L) + u * _L
                    xs.append(
                        (
                            buf[j % 2, 0, rr, pl.ds(o, _L)],
                            buf[j % 2, 1, rr, pl.ds(o, _L)],
                            buf[j % 2, 2, rr, pl.ds(o, _L)],
                        )
                    )
                addrs = [
                    lane_base + (((xr & 0xF0) << 4) | (xg & 0xF0) | (xb >> 4))
                    for (xb, xg, xr) in xs
                ]
                for a in addrs:
                    plsc.addupdate_scatter(hist, [a], ones)
                return carry

            lax.fori_loop(0, _WIDTH // (_U * _L), it, 0)

        if j + 2 < _NCHUNK:
            descs[j % 2] = start(j + 2)

    def rbody(i, carry):
        for u in range(2):
            o = i * (2 * _L) + u * _L
            vs = [hist[pl.ds(l * _NBINS + o, _L)] for l in range(_L)]
            while len(vs) > 1:
                vs = [a + b for a, b in zip(vs[0::2], vs[1::2])]
            hist[pl.ds(o, _L)] = vs[0]
        return carry

    lax.fori_loop(0, _NBINS // (2 * _L), rbody, 0)
    pltpu.sync_copy(hist.at[pl.ds(0, _NBINS)], out_hbm.at[wid])


def _reduce_tc_body(parts_ref, out_ref):
    s = jnp.sum(parts_ref[...], axis=0, keepdims=True)
    body = jnp.broadcast_to(s, (8, _NBINS))
    sent = (lax.broadcasted_iota(jnp.int32, (8, 8), 1) == 0).astype(jnp.int32)
    out_ref[...] = jnp.concatenate([body, sent], axis=1)


_reduce = pl.pallas_call(
    _reduce_tc_body,
    out_shape=jax.ShapeDtypeStruct((8, _NBINS + 8), jnp.int32),
)


@jax.jit
def kernel(img):
    planar = jnp.transpose(img.astype(jnp.int32), (2, 0, 1))
    parts = _hist(planar)
    full = _reduce(parts)
    return full[0, : _NBINS + 1]
